# baseline (speedup 1.0000x reference)
import functools

import jax
import jax.numpy as jnp
from jax import lax
from jax.experimental import pallas as pl
from jax.experimental.pallas import tpu as pltpu
from jax.experimental.pallas import tpu_sc as plsc

B, S, H = 2, 2048, 1024
V = 32000
E = 8
SMALL = 256
N_EXPERT_LAYERS = 2
N_MODEL_LAYERS = 4
EPS = 1.1920929e-07

BT = B * S
NC, NS = 2, 16
NW = NC * NS
ROWS_PER_W = BT // NW
CHUNK = 32


def _embed_gather(embed, tok):
    mesh = plsc.VectorSubcoreMesh(core_axis_name="c", subcore_axis_name="s")

    @functools.partial(
        pl.kernel,
        mesh=mesh,
        out_type=jax.ShapeDtypeStruct((BT, H), jnp.float32),
        scratch_types=[
            pltpu.VMEM((CHUNK,), jnp.int32),
            pltpu.VMEM((CHUNK, H), jnp.float32),
            pltpu.SemaphoreType.DMA,
        ],
    )
    def gather_k(table_hbm, idx_hbm, out_hbm, idx_v, rows_v, sem):
        wid = lax.axis_index("s") * NC + lax.axis_index("c")
        base = wid * ROWS_PER_W
        for c in range(ROWS_PER_W // CHUNK):
            off = base + c * CHUNK
            pltpu.sync_copy(idx_hbm.at[pl.ds(off, CHUNK)], idx_v)
            pltpu.async_copy(table_hbm.at[idx_v], rows_v, sem).wait()
            pltpu.sync_copy(rows_v, out_hbm.at[pl.ds(off, CHUNK)])

    return gather_k(embed, tok)


def _router_body(x_ref, rw_ref, idx_ref, w_ref):
    x = x_ref[...]
    s0 = jnp.sum(x[0:S, :], axis=0, keepdims=True)
    s1 = jnp.sum(x[S:, :], axis=0, keepdims=True)
    xm = jnp.concatenate([s0, s1], axis=0) * (1.0 / S)
    logits = 2.0 * lax.dot_general(
        xm, rw_ref[...], (((1,), (1,)), ((), ())),
        preferred_element_type=jnp.float32)
    m = jnp.max(logits, axis=1, keepdims=True)
    e = jnp.exp(logits - m)
    p = e / jnp.sum(e, axis=1, keepdims=True)
    iota = lax.broadcasted_iota(jnp.int32, (B, E), 1)
    v1 = jnp.max(p, axis=1, keepdims=True)
    i1 = jnp.min(jnp.where(p >= v1, iota, E), axis=1, keepdims=True)
    p2 = jnp.where(iota == i1, -1.0, p)
    v2 = jnp.max(p2, axis=1, keepdims=True)
    i2 = jnp.min(jnp.where(p2 >= v2, iota, E), axis=1, keepdims=True)
    mm = jnp.maximum(v1, v2)
    a = jnp.exp(v1 - mm)
    b = jnp.exp(v2 - mm)
    denom = a + b
    idx_ref[...] = jnp.concatenate([i1, i2], axis=1)
    w_ref[...] = jnp.concatenate([a / denom, b / denom], axis=1)


def _router_call(x, router_w):
    return pl.pallas_call(
        _router_body,
        out_shape=(
            jax.ShapeDtypeStruct((B, 2), jnp.int32),
            jax.ShapeDtypeStruct((B, 2), jnp.float32),
        ),
    )(x, router_w)


S_TILE = 1024
N_S_TILES = BT // S_TILE
TILES_PER_B = S // S_TILE


def _silu(z):
    return z * lax.logistic(z)


def _mlp_pair(x, w1, w2):
    h = _silu(lax.dot_general(x.astype(jnp.bfloat16), w1.astype(jnp.bfloat16),
                              (((1,), (1,)), ((), ())),
                              preferred_element_type=jnp.float32))
    d = lax.dot_general(h.astype(jnp.bfloat16), w2.astype(jnp.bfloat16),
                        (((1,), (1,)), ((), ())),
                        preferred_element_type=jnp.float32)
    return x + 0.1 * d


def _mega_body(idx_ref, x_ref, bb1_ref, bb2_ref, ew1_ref, ew2_ref,
               w2_ref, nw_ref, out_ref, xb_ref):
    k = pl.program_id(1)
    bidx = pl.program_id(0) // TILES_PER_B

    @pl.when(k == 0)
    def _():
        xb = x_ref[...]
        for i in range(N_MODEL_LAYERS):
            xb = _mlp_pair(xb, bb1_ref[i], bb2_ref[i])
        xb_ref[...] = xb

    xb = xb_ref[...]
    xe = xb
    for l in range(N_EXPERT_LAYERS):
        xe = _mlp_pair(xe, ew1_ref[0, l], ew2_ref[0, l])
    w = w2_ref[bidx, k]

    @pl.when(k == 0)
    def _():
        out_ref[...] = (w * xe).astype(out_ref.dtype)

    @pl.when(k == 1)
    def _():
        y = xb + out_ref[...].astype(jnp.float32) + w * xe
        var = jnp.mean(y * y, axis=-1, keepdims=True)
        yn = y * lax.rsqrt(var + EPS) * nw_ref[...]
        out_ref[...] = yn.astype(out_ref.dtype)


def _mega_call(idx_flat, x, bb_w1, bb_w2, ex_w1, ex_w2, top2_w, norm_w):
    grid_spec = pltpu.PrefetchScalarGridSpec(
        num_scalar_prefetch=1,
        grid=(N_S_TILES, 2),
        in_specs=[
            pl.BlockSpec((S_TILE, H), lambda s, k, idx: (s, 0)),
            pl.BlockSpec((N_MODEL_LAYERS, SMALL, H), lambda s, k, idx: (0, 0, 0)),
            pl.BlockSpec((N_MODEL_LAYERS, H, SMALL), lambda s, k, idx: (0, 0, 0)),
            pl.BlockSpec((1, N_EXPERT_LAYERS, SMALL, H),
                         lambda s, k, idx: (idx[2 * (s // TILES_PER_B) + k], 0, 0, 0)),
            pl.BlockSpec((1, N_EXPERT_LAYERS, H, SMALL),
                         lambda s, k, idx: (idx[2 * (s // TILES_PER_B) + k], 0, 0, 0)),
            pl.BlockSpec(memory_space=pltpu.SMEM),
            pl.BlockSpec((1, H), lambda s, k, idx: (0, 0)),
        ],
        out_specs=pl.BlockSpec((S_TILE, H), lambda s, k, idx: (s, 0)),
        scratch_shapes=[pltpu.VMEM((S_TILE, H), jnp.float32)],
    )
    return pl.pallas_call(
        _mega_body,
        grid_spec=grid_spec,
        out_shape=jax.ShapeDtypeStruct((BT, H), jnp.bfloat16),
        compiler_params=pltpu.CompilerParams(
            dimension_semantics=("arbitrary", "arbitrary")),
    )(idx_flat, x, bb_w1, bb_w2, ex_w1, ex_w2, top2_w,
      norm_w.reshape(1, H))


V_TILE = 1280
N_V_TILES = V // V_TILE
HS_TILE = 2048
N_HS_TILES = BT // HS_TILE


def _head_body(x_ref, w_ref, out_ref):
    s = pl.program_id(1)
    xs = x_ref[pl.ds(s * HS_TILE, HS_TILE), :]
    wb = w_ref[...].astype(jnp.bfloat16)
    out_ref[0] = lax.dot_general(
        xs, wb, (((1,), (1,)), ((), ())),
        preferred_element_type=jnp.float32)


def _head_call(xn, head_w):
    return pl.pallas_call(
        _head_body,
        grid=(N_V_TILES, N_HS_TILES),
        in_specs=[
            pl.BlockSpec((BT, H), lambda v, s: (0, 0)),
            pl.BlockSpec((V_TILE, H), lambda v, s: (v, 0)),
        ],
        out_specs=pl.BlockSpec((1, HS_TILE, V_TILE), lambda v, s: (s, 0, v)),
        out_shape=jax.ShapeDtypeStruct((B, S, V), jnp.float32),
        compiler_params=pltpu.CompilerParams(
            dimension_semantics=("arbitrary", "arbitrary")),
    )(xn, head_w)


def kernel(tokens, embed, router_w, bb_w1, bb_w2, ex_w1, ex_w2, norm_w, head_w):
    tok = tokens.reshape(-1).astype(jnp.int32)
    x = _embed_gather(embed, tok)
    top2_idx, top2_w = _router_call(x, router_w)
    idx_flat = top2_idx.reshape(-1)
    xn = _mega_call(idx_flat, x, bb_w1, bb_w2, ex_w1, ex_w2,
                    top2_w, norm_w)
    return _head_call(xn, head_w)


# speedup vs baseline: 1.0353x; 1.0353x over previous
import functools

import jax
import jax.numpy as jnp
from jax import lax
from jax.experimental import pallas as pl
from jax.experimental.pallas import tpu as pltpu
from jax.experimental.pallas import tpu_sc as plsc

B, S, H = 2, 2048, 1024
V = 32000
E = 8
SMALL = 256
N_EXPERT_LAYERS = 2
N_MODEL_LAYERS = 4
EPS = 1.1920929e-07

BT = B * S
NC, NS = 2, 16
NW = NC * NS
ROWS_PER_W = BT // NW
CHUNK = 32


def _embed_gather(embed, tok):
    mesh = plsc.VectorSubcoreMesh(core_axis_name="c", subcore_axis_name="s")

    @functools.partial(
        pl.kernel,
        mesh=mesh,
        out_type=jax.ShapeDtypeStruct((BT, H), jnp.float32),
        scratch_types=[
            pltpu.VMEM((ROWS_PER_W,), jnp.int32),
            pltpu.VMEM((2, CHUNK, H), jnp.float32),
            pltpu.SemaphoreType.DMA,
            pltpu.SemaphoreType.DMA,
        ],
    )
    def gather_k(table_hbm, idx_hbm, out_hbm, idx_v, rows_v, sem0, sem1):
        wid = lax.axis_index("s") * NC + lax.axis_index("c")
        base = wid * ROWS_PER_W
        n_chunks = ROWS_PER_W // CHUNK
        sems = [sem0, sem1]
        pltpu.sync_copy(idx_hbm.at[pl.ds(base, ROWS_PER_W)], idx_v)
        cp = pltpu.async_copy(table_hbm.at[idx_v.at[pl.ds(0, CHUNK)]],
                              rows_v.at[0], sems[0])
        for c in range(n_chunks):
            cp.wait()
            if c + 1 < n_chunks:
                cp = pltpu.async_copy(
                    table_hbm.at[idx_v.at[pl.ds((c + 1) * CHUNK, CHUNK)]],
                    rows_v.at[(c + 1) % 2], sems[(c + 1) % 2])
            pltpu.sync_copy(rows_v.at[c % 2],
                            out_hbm.at[pl.ds(base + c * CHUNK, CHUNK)])

    return gather_k(embed, tok)


def _router_body(x_ref, rw_ref, idx_ref, w_ref):
    x = x_ref[...]
    s0 = jnp.sum(x[0:S, :], axis=0, keepdims=True)
    s1 = jnp.sum(x[S:, :], axis=0, keepdims=True)
    xm = jnp.concatenate([s0, s1], axis=0) * (1.0 / S)
    logits = 2.0 * lax.dot_general(
        xm, rw_ref[...], (((1,), (1,)), ((), ())),
        preferred_element_type=jnp.float32)
    m = jnp.max(logits, axis=1, keepdims=True)
    e = jnp.exp(logits - m)
    p = e / jnp.sum(e, axis=1, keepdims=True)
    iota = lax.broadcasted_iota(jnp.int32, (B, E), 1)
    v1 = jnp.max(p, axis=1, keepdims=True)
    i1 = jnp.min(jnp.where(p >= v1, iota, E), axis=1, keepdims=True)
    p2 = jnp.where(iota == i1, -1.0, p)
    v2 = jnp.max(p2, axis=1, keepdims=True)
    i2 = jnp.min(jnp.where(p2 >= v2, iota, E), axis=1, keepdims=True)
    mm = jnp.maximum(v1, v2)
    a = jnp.exp(v1 - mm)
    b = jnp.exp(v2 - mm)
    denom = a + b
    idx_ref[...] = jnp.concatenate([i1, i2], axis=1)
    w_ref[...] = jnp.concatenate([a / denom, b / denom], axis=1)


def _router_call(x, router_w):
    return pl.pallas_call(
        _router_body,
        out_shape=(
            jax.ShapeDtypeStruct((B, 2), jnp.int32),
            jax.ShapeDtypeStruct((B, 2), jnp.float32),
        ),
    )(x, router_w)


S_TILE = 1024
N_S_TILES = BT // S_TILE
TILES_PER_B = S // S_TILE


def _silu(z):
    return z * lax.logistic(z)


def _mlp_pair(x, w1, w2):
    h = _silu(lax.dot_general(x.astype(jnp.bfloat16), w1.astype(jnp.bfloat16),
                              (((1,), (1,)), ((), ())),
                              preferred_element_type=jnp.float32))
    d = lax.dot_general((0.1 * h).astype(jnp.bfloat16), w2.astype(jnp.bfloat16),
                        (((1,), (1,)), ((), ())),
                        preferred_element_type=jnp.float32)
    return x + d


def _mega_body(idx_ref, x_ref, bb1_ref, bb2_ref, ew1a_ref, ew2a_ref,
               ew1b_ref, ew2b_ref, w2_ref, nw_ref, out_ref):
    bidx = pl.program_id(0) // TILES_PER_B
    xb = x_ref[...]
    for i in range(N_MODEL_LAYERS):
        xb = _mlp_pair(xb, bb1_ref[i], bb2_ref[i])
    xe0 = xb
    xe1 = xb
    for l in range(N_EXPERT_LAYERS):
        xe0 = _mlp_pair(xe0, ew1a_ref[0, l], ew2a_ref[0, l])
    for l in range(N_EXPERT_LAYERS):
        xe1 = _mlp_pair(xe1, ew1b_ref[0, l], ew2b_ref[0, l])
    w0 = w2_ref[bidx, 0]
    w1 = w2_ref[bidx, 1]
    y = xb + w0 * xe0 + w1 * xe1
    var = jnp.mean(y * y, axis=-1, keepdims=True)
    yn = y * lax.rsqrt(var + EPS) * nw_ref[...]
    out_ref[...] = yn.astype(out_ref.dtype)


def _mega_call(idx_flat, x, bb_w1, bb_w2, ex_w1, ex_w2, top2_w, norm_w):
    ew1_spec0 = pl.BlockSpec(
        (1, N_EXPERT_LAYERS, SMALL, H),
        lambda s, idx: (idx[2 * (s // TILES_PER_B)], 0, 0, 0))
    ew2_spec0 = pl.BlockSpec(
        (1, N_EXPERT_LAYERS, H, SMALL),
        lambda s, idx: (idx[2 * (s // TILES_PER_B)], 0, 0, 0))
    ew1_spec1 = pl.BlockSpec(
        (1, N_EXPERT_LAYERS, SMALL, H),
        lambda s, idx: (idx[2 * (s // TILES_PER_B) + 1], 0, 0, 0))
    ew2_spec1 = pl.BlockSpec(
        (1, N_EXPERT_LAYERS, H, SMALL),
        lambda s, idx: (idx[2 * (s // TILES_PER_B) + 1], 0, 0, 0))
    grid_spec = pltpu.PrefetchScalarGridSpec(
        num_scalar_prefetch=1,
        grid=(N_S_TILES,),
        in_specs=[
            pl.BlockSpec((S_TILE, H), lambda s, idx: (s, 0)),
            pl.BlockSpec((N_MODEL_LAYERS, SMALL, H), lambda s, idx: (0, 0, 0)),
            pl.BlockSpec((N_MODEL_LAYERS, H, SMALL), lambda s, idx: (0, 0, 0)),
            ew1_spec0, ew2_spec0, ew1_spec1, ew2_spec1,
            pl.BlockSpec(memory_space=pltpu.SMEM),
            pl.BlockSpec((1, H), lambda s, idx: (0, 0)),
        ],
        out_specs=pl.BlockSpec((S_TILE, H), lambda s, idx: (s, 0)),
    )
    return pl.pallas_call(
        _mega_body,
        grid_spec=grid_spec,
        out_shape=jax.ShapeDtypeStruct((BT, H), jnp.bfloat16),
        compiler_params=pltpu.CompilerParams(
            dimension_semantics=("arbitrary",)),
    )(idx_flat, x, bb_w1, bb_w2, ex_w1, ex_w2, ex_w1, ex_w2, top2_w,
      norm_w.reshape(1, H))


V_TILE = 1280
N_V_TILES = V // V_TILE
HS_TILE = 2048
N_HS_TILES = BT // HS_TILE


def _head_body(x_ref, w_ref, out_ref):
    s = pl.program_id(1)
    xs = x_ref[pl.ds(s * HS_TILE, HS_TILE), :]
    wb = w_ref[...].astype(jnp.bfloat16)
    out_ref[0] = lax.dot_general(
        xs, wb, (((1,), (1,)), ((), ())),
        preferred_element_type=jnp.float32)


def _head_call(xn, head_w):
    return pl.pallas_call(
        _head_body,
        grid=(N_V_TILES, N_HS_TILES),
        in_specs=[
            pl.BlockSpec((BT, H), lambda v, s: (0, 0)),
            pl.BlockSpec((V_TILE, H), lambda v, s: (v, 0)),
        ],
        out_specs=pl.BlockSpec((1, HS_TILE, V_TILE), lambda v, s: (s, 0, v)),
        out_shape=jax.ShapeDtypeStruct((B, S, V), jnp.float32),
        compiler_params=pltpu.CompilerParams(
            dimension_semantics=("arbitrary", "arbitrary")),
    )(xn, head_w)


def kernel(tokens, embed, router_w, bb_w1, bb_w2, ex_w1, ex_w2, norm_w, head_w):
    tok = tokens.reshape(-1).astype(jnp.int32)
    x = _embed_gather(embed, tok)
    top2_idx, top2_w = _router_call(x, router_w)
    idx_flat = top2_idx.reshape(-1)
    xn = _mega_call(idx_flat, x, bb_w1, bb_w2, ex_w1, ex_w2,
                    top2_w, norm_w)
    return _head_call(xn, head_w)


# speedup vs baseline: 1.0361x; 1.0007x over previous
import functools

import jax
import jax.numpy as jnp
from jax import lax
from jax.experimental import pallas as pl
from jax.experimental.pallas import tpu as pltpu
from jax.experimental.pallas import tpu_sc as plsc

B, S, H = 2, 2048, 1024
V = 32000
E = 8
SMALL = 256
N_EXPERT_LAYERS = 2
N_MODEL_LAYERS = 4
EPS = 1.1920929e-07

BT = B * S
NC, NS = 2, 16
NW = NC * NS
ROWS_PER_W = BT // NW
CHUNK = 32


def _embed_gather(embed, tok):
    mesh = plsc.VectorSubcoreMesh(core_axis_name="c", subcore_axis_name="s")

    @functools.partial(
        pl.kernel,
        mesh=mesh,
        out_type=jax.ShapeDtypeStruct((BT, H), jnp.float32),
        scratch_types=[
            pltpu.VMEM((ROWS_PER_W,), jnp.int32),
            pltpu.VMEM((2, CHUNK, H), jnp.float32),
            pltpu.SemaphoreType.DMA,
            pltpu.SemaphoreType.DMA,
        ],
    )
    def gather_k(table_hbm, idx_hbm, out_hbm, idx_v, rows_v, sem0, sem1):
        wid = lax.axis_index("s") * NC + lax.axis_index("c")
        base = wid * ROWS_PER_W
        n_chunks = ROWS_PER_W // CHUNK
        sems = [sem0, sem1]
        pltpu.sync_copy(idx_hbm.at[pl.ds(base, ROWS_PER_W)], idx_v)
        cp = pltpu.async_copy(table_hbm.at[idx_v.at[pl.ds(0, CHUNK)]],
                              rows_v.at[0], sems[0])
        for c in range(n_chunks):
            cp.wait()
            if c + 1 < n_chunks:
                cp = pltpu.async_copy(
                    table_hbm.at[idx_v.at[pl.ds((c + 1) * CHUNK, CHUNK)]],
                    rows_v.at[(c + 1) % 2], sems[(c + 1) % 2])
            pltpu.sync_copy(rows_v.at[c % 2],
                            out_hbm.at[pl.ds(base + c * CHUNK, CHUNK)])

    return gather_k(embed, tok)


RT_TILE = 1024
N_RT_TILES = BT // RT_TILE
RT_PER_B = S // RT_TILE


def _router_body(x_ref, rw_ref, idx_ref, w_ref, sum_ref):
    s = pl.program_id(0)
    b = s // RT_PER_B
    ps = jnp.sum(x_ref[...], axis=0, keepdims=True)

    @pl.when(s % RT_PER_B == 0)
    def _():
        sum_ref[pl.ds(b, 1), :] = ps

    @pl.when(s % RT_PER_B != 0)
    def _():
        sum_ref[pl.ds(b, 1), :] = sum_ref[pl.ds(b, 1), :] + ps

    @pl.when(s == N_RT_TILES - 1)
    def _():
        _router_finish(rw_ref, idx_ref, w_ref, sum_ref)


def _router_finish(rw_ref, idx_ref, w_ref, sum_ref):
    xm = sum_ref[...] * (1.0 / S)
    logits = 2.0 * lax.dot_general(
        xm, rw_ref[...], (((1,), (1,)), ((), ())),
        preferred_element_type=jnp.float32)
    m = jnp.max(logits, axis=1, keepdims=True)
    e = jnp.exp(logits - m)
    p = e / jnp.sum(e, axis=1, keepdims=True)
    iota = lax.broadcasted_iota(jnp.int32, (B, E), 1)
    v1 = jnp.max(p, axis=1, keepdims=True)
    i1 = jnp.min(jnp.where(p >= v1, iota, E), axis=1, keepdims=True)
    p2 = jnp.where(iota == i1, -1.0, p)
    v2 = jnp.max(p2, axis=1, keepdims=True)
    i2 = jnp.min(jnp.where(p2 >= v2, iota, E), axis=1, keepdims=True)
    mm = jnp.maximum(v1, v2)
    a = jnp.exp(v1 - mm)
    b = jnp.exp(v2 - mm)
    denom = a + b
    idx_ref[...] = jnp.concatenate([i1, i2], axis=1)
    w_ref[...] = jnp.concatenate([a / denom, b / denom], axis=1)


def _router_call(x, router_w):
    return pl.pallas_call(
        _router_body,
        grid=(N_RT_TILES,),
        in_specs=[
            pl.BlockSpec((RT_TILE, H), lambda s: (s, 0)),
            pl.BlockSpec((E, H), lambda s: (0, 0)),
        ],
        out_specs=(
            pl.BlockSpec((B, 2), lambda s: (0, 0)),
            pl.BlockSpec((B, 2), lambda s: (0, 0)),
        ),
        out_shape=(
            jax.ShapeDtypeStruct((B, 2), jnp.int32),
            jax.ShapeDtypeStruct((B, 2), jnp.float32),
        ),
        scratch_shapes=[pltpu.VMEM((B, H), jnp.float32)],
        compiler_params=pltpu.CompilerParams(
            dimension_semantics=("arbitrary",)),
    )(x, router_w)


S_TILE = 1024
N_S_TILES = BT // S_TILE
TILES_PER_B = S // S_TILE


def _silu(z):
    return z * lax.logistic(z)


def _mlp_pair(x, w1, w2):
    h = _silu(lax.dot_general(x.astype(jnp.bfloat16), w1.astype(jnp.bfloat16),
                              (((1,), (1,)), ((), ())),
                              preferred_element_type=jnp.float32))
    d = lax.dot_general((0.1 * h).astype(jnp.bfloat16), w2.astype(jnp.bfloat16),
                        (((1,), (1,)), ((), ())),
                        preferred_element_type=jnp.float32)
    return x + d


def _mega_body(idx_ref, x_ref, bb1_ref, bb2_ref, ew1a_ref, ew2a_ref,
               ew1b_ref, ew2b_ref, w2_ref, nw_ref, out_ref):
    bidx = pl.program_id(0) // TILES_PER_B
    xb = x_ref[...]
    for i in range(N_MODEL_LAYERS):
        xb = _mlp_pair(xb, bb1_ref[i], bb2_ref[i])
    xe0 = xb
    xe1 = xb
    for l in range(N_EXPERT_LAYERS):
        xe0 = _mlp_pair(xe0, ew1a_ref[0, l], ew2a_ref[0, l])
    for l in range(N_EXPERT_LAYERS):
        xe1 = _mlp_pair(xe1, ew1b_ref[0, l], ew2b_ref[0, l])
    w0 = w2_ref[bidx, 0]
    w1 = w2_ref[bidx, 1]
    y = xb + w0 * xe0 + w1 * xe1
    var = jnp.mean(y * y, axis=-1, keepdims=True)
    yn = y * lax.rsqrt(var + EPS) * nw_ref[...]
    out_ref[...] = yn.astype(out_ref.dtype)


def _mega_call(idx_flat, x, bb_w1, bb_w2, ex_w1, ex_w2, top2_w, norm_w):
    ew1_spec0 = pl.BlockSpec(
        (1, N_EXPERT_LAYERS, SMALL, H),
        lambda s, idx: (idx[2 * (s // TILES_PER_B)], 0, 0, 0))
    ew2_spec0 = pl.BlockSpec(
        (1, N_EXPERT_LAYERS, H, SMALL),
        lambda s, idx: (idx[2 * (s // TILES_PER_B)], 0, 0, 0))
    ew1_spec1 = pl.BlockSpec(
        (1, N_EXPERT_LAYERS, SMALL, H),
        lambda s, idx: (idx[2 * (s // TILES_PER_B) + 1], 0, 0, 0))
    ew2_spec1 = pl.BlockSpec(
        (1, N_EXPERT_LAYERS, H, SMALL),
        lambda s, idx: (idx[2 * (s // TILES_PER_B) + 1], 0, 0, 0))
    grid_spec = pltpu.PrefetchScalarGridSpec(
        num_scalar_prefetch=1,
        grid=(N_S_TILES,),
        in_specs=[
            pl.BlockSpec((S_TILE, H), lambda s, idx: (s, 0)),
            pl.BlockSpec((N_MODEL_LAYERS, SMALL, H), lambda s, idx: (0, 0, 0)),
            pl.BlockSpec((N_MODEL_LAYERS, H, SMALL), lambda s, idx: (0, 0, 0)),
            ew1_spec0, ew2_spec0, ew1_spec1, ew2_spec1,
            pl.BlockSpec(memory_space=pltpu.SMEM),
            pl.BlockSpec((1, H), lambda s, idx: (0, 0)),
        ],
        out_specs=pl.BlockSpec((S_TILE, H), lambda s, idx: (s, 0)),
    )
    return pl.pallas_call(
        _mega_body,
        grid_spec=grid_spec,
        out_shape=jax.ShapeDtypeStruct((BT, H), jnp.bfloat16),
        compiler_params=pltpu.CompilerParams(
            dimension_semantics=("arbitrary",)),
    )(idx_flat, x, bb_w1, bb_w2, ex_w1, ex_w2, ex_w1, ex_w2, top2_w,
      norm_w.reshape(1, H))


V_TILE = 1280
N_V_TILES = V // V_TILE
HS_TILE = 2048
N_HS_TILES = BT // HS_TILE


def _head_body(x_ref, w_ref, out_ref):
    s = pl.program_id(1)
    xs = x_ref[pl.ds(s * HS_TILE, HS_TILE), :]
    wb = w_ref[...].astype(jnp.bfloat16)
    out_ref[0] = lax.dot_general(
        xs, wb, (((1,), (1,)), ((), ())),
        preferred_element_type=jnp.float32)


def _head_call(xn, head_w):
    return pl.pallas_call(
        _head_body,
        grid=(N_V_TILES, N_HS_TILES),
        in_specs=[
            pl.BlockSpec((BT, H), lambda v, s: (0, 0)),
            pl.BlockSpec((V_TILE, H), lambda v, s: (v, 0)),
        ],
        out_specs=pl.BlockSpec((1, HS_TILE, V_TILE), lambda v, s: (s, 0, v)),
        out_shape=jax.ShapeDtypeStruct((B, S, V), jnp.float32),
        compiler_params=pltpu.CompilerParams(
            dimension_semantics=("arbitrary", "arbitrary")),
    )(xn, head_w)


def kernel(tokens, embed, router_w, bb_w1, bb_w2, ex_w1, ex_w2, norm_w, head_w):
    tok = tokens.reshape(-1).astype(jnp.int32)
    x = _embed_gather(embed, tok)
    top2_idx, top2_w = _router_call(x, router_w)
    idx_flat = top2_idx.reshape(-1)
    xn = _mega_call(idx_flat, x, bb_w1, bb_w2, ex_w1, ex_w2,
                    top2_w, norm_w)
    return _head_call(xn, head_w)
